# Initial kernel scaffold; baseline (speedup 1.0000x reference)
import functools

import jax
import jax.numpy as jnp
from jax import lax
from jax.experimental import pallas as pl
from jax.experimental.pallas import tpu as pltpu
from jax.experimental.pallas import tpu_sc as plsc

D = 32
B_TOTAL = 16384 * 50
NC = 2
NS = 16
NW = NC * NS
B_PER_W = B_TOTAL // NW
C = 128
N_CHUNKS = B_PER_W // C
NB = 4
N_OUTER = N_CHUNKS // NB

_mesh = plsc.VectorSubcoreMesh(core_axis_name="c", subcore_axis_name="s")


@functools.partial(
    pl.kernel,
    out_type=jax.ShapeDtypeStruct((B_TOTAL, D), jnp.float32),
    mesh=_mesh,
    scratch_types=[
        pltpu.VMEM((N_CHUNKS, C), jnp.int32),
        pltpu.VMEM((NB, C, D), jnp.float32),
        pltpu.SemaphoreType.DMA,
        pltpu.SemaphoreType.DMA,
        pltpu.SemaphoreType.DMA,
        pltpu.SemaphoreType.DMA,
    ],
)
def _gather(idx_hbm, table_hbm, out_hbm, idx_v, rows_v, s0, s1, s2, s3):
    sems = (s0, s1, s2, s3)
    wid = lax.axis_index("s") * NC + lax.axis_index("c")
    base = wid * B_PER_W

    pltpu.sync_copy(idx_hbm.at[pl.ds(wid * N_CHUNKS, N_CHUNKS)], idx_v)

    for b in range(NB):
        pltpu.async_copy(table_hbm.at[idx_v.at[b]], rows_v.at[b], sems[b])

    def turn(g, carry):
        for b in range(NB):
            j = g * NB + b
            pltpu.make_async_copy(
                table_hbm.at[idx_v.at[j]], rows_v.at[b], sems[b]
            ).wait()
            pltpu.sync_copy(rows_v.at[b], out_hbm.at[pl.ds(base + j * C, C)])
            pltpu.async_copy(table_hbm.at[idx_v.at[j + NB]], rows_v.at[b], sems[b])
        return carry

    lax.fori_loop(0, N_OUTER - 1, turn, 0)

    for b in range(NB):
        j = (N_OUTER - 1) * NB + b
        pltpu.make_async_copy(
            table_hbm.at[idx_v.at[j]], rows_v.at[b], sems[b]
        ).wait()
        pltpu.sync_copy(rows_v.at[b], out_hbm.at[pl.ds(base + j * C, C)])


def kernel(input, weight):
    batch, hist = input.shape
    idx = input.astype(jnp.int32).reshape(NW * N_CHUNKS, C)
    out = _gather(idx, weight)
    return out.reshape(batch, hist, D)


# speedup vs baseline: 1.1095x; 1.1095x over previous
import functools

import jax
import jax.numpy as jnp
from jax import lax
from jax.experimental import pallas as pl
from jax.experimental.pallas import tpu as pltpu
from jax.experimental.pallas import tpu_sc as plsc

D = 32
B_TOTAL = 16384 * 50
NC = 2
NS = 16
NW = NC * NS
B_PER_W = B_TOTAL // NW
C = 128
N_CHUNKS = B_PER_W // C
NB = 4
N_OUTER = N_CHUNKS // NB

_mesh = plsc.VectorSubcoreMesh(core_axis_name="c", subcore_axis_name="s")


@functools.partial(
    pl.kernel,
    out_type=jax.ShapeDtypeStruct((B_TOTAL, D), jnp.float32),
    mesh=_mesh,
    compiler_params=pltpu.CompilerParams(use_tc_tiling_on_sc=False),
    scratch_types=[
        pltpu.VMEM((N_CHUNKS, C), jnp.int32),
        pltpu.VMEM((NB, C, D), jnp.float32),
        pltpu.SemaphoreType.DMA,
        pltpu.SemaphoreType.DMA,
        pltpu.SemaphoreType.DMA,
        pltpu.SemaphoreType.DMA,
    ],
)
def _gather(idx_hbm, table_hbm, out_hbm, idx_v, rows_v, s0, s1, s2, s3):
    sems = (s0, s1, s2, s3)
    wid = lax.axis_index("s") * NC + lax.axis_index("c")
    base = wid * B_PER_W

    pltpu.sync_copy(idx_hbm.at[pl.ds(wid * N_CHUNKS, N_CHUNKS)], idx_v)

    for b in range(NB):
        pltpu.async_copy(table_hbm.at[idx_v.at[b]], rows_v.at[b], sems[b])

    def turn(g, carry):
        for b in range(NB):
            j = g * NB + b
            pltpu.make_async_copy(
                table_hbm.at[idx_v.at[j]], rows_v.at[b], sems[b]
            ).wait()
            pltpu.sync_copy(rows_v.at[b], out_hbm.at[pl.ds(base + j * C, C)])
            pltpu.async_copy(table_hbm.at[idx_v.at[j + NB]], rows_v.at[b], sems[b])
        return carry

    lax.fori_loop(0, N_OUTER - 1, turn, 0)

    for b in range(NB):
        j = (N_OUTER - 1) * NB + b
        pltpu.make_async_copy(
            table_hbm.at[idx_v.at[j]], rows_v.at[b], sems[b]
        ).wait()
        pltpu.sync_copy(rows_v.at[b], out_hbm.at[pl.ds(base + j * C, C)])


def kernel(input, weight):
    batch, hist = input.shape
    idx = input.astype(jnp.int32).reshape(NW * N_CHUNKS, C)
    out = _gather(idx, weight)
    return out.reshape(batch, hist, D)


# speedup vs baseline: 1.1135x; 1.0036x over previous
import functools

import jax
import jax.numpy as jnp
from jax import lax
from jax.experimental import pallas as pl
from jax.experimental.pallas import tpu as pltpu
from jax.experimental.pallas import tpu_sc as plsc

D = 32
B_TOTAL = 16384 * 50
NC = 2
NS = 16
NW = NC * NS
B_PER_W = B_TOTAL // NW
C = 128
N_CHUNKS = B_PER_W // C
S = 8
N_ROUNDS = N_CHUNKS // S

_mesh = plsc.VectorSubcoreMesh(core_axis_name="c", subcore_axis_name="s")


@functools.partial(
    pl.kernel,
    out_type=jax.ShapeDtypeStruct((B_TOTAL, D), jnp.float32),
    mesh=_mesh,
    compiler_params=pltpu.CompilerParams(use_tc_tiling_on_sc=False),
    scratch_types=[
        pltpu.VMEM((N_CHUNKS, C), jnp.int32),
        pltpu.VMEM((S, 2, C, D), jnp.float32),
        pltpu.SemaphoreType.DMA((S,)),
        pltpu.SemaphoreType.DMA((S,)),
    ],
)
def _gather(idx_hbm, table_hbm, out_hbm, idx_v, buf, gsem, ssem):
    wid = lax.axis_index("s") * NC + lax.axis_index("c")
    base = wid * B_PER_W

    pltpu.sync_copy(idx_hbm.at[pl.ds(wid * N_CHUNKS, N_CHUNKS)], idx_v)

    def fire(j, half, s):
        pltpu.async_copy(table_hbm.at[idx_v.at[j]], buf.at[s].at[half], gsem.at[s])

    def wait_gather(j, half, s):
        pltpu.make_async_copy(
            table_hbm.at[idx_v.at[j]], buf.at[s].at[half], gsem.at[s]
        ).wait()

    def store(j, half, s):
        pltpu.async_copy(
            buf.at[s].at[half], out_hbm.at[pl.ds(base + j * C, C)], ssem.at[s]
        )

    def drain_one_store(half, s):
        pltpu.make_async_copy(
            table_hbm.at[pl.ds(0, C)], buf.at[s].at[half], ssem.at[s]
        ).wait()

    for s in range(S):
        fire(s, 0, s)

    for s in range(S):
        wait_gather(s, 0, s)
        store(s, 0, s)
        fire(s + S, 1, s)

    def round_body(r, carry):
        h = r % 2
        hn = 1 - h
        for s in range(S):
            j = r * S + s
            wait_gather(j, h, s)
            store(j, h, s)
            drain_one_store(hn, s)
            fire(j + S, hn, s)
        return carry

    lax.fori_loop(1, N_ROUNDS - 1, round_body, 0)

    hl = (N_ROUNDS - 1) % 2
    for s in range(S):
        j = (N_ROUNDS - 1) * S + s
        wait_gather(j, hl, s)
        store(j, hl, s)

    for s in range(S):
        drain_one_store(0, s)
        drain_one_store(1, s)


def kernel(input, weight):
    batch, hist = input.shape
    idx = input.astype(jnp.int32).reshape(NW * N_CHUNKS, C)
    out = _gather(idx, weight)
    return out.reshape(batch, hist, D)


# speedup vs baseline: 1.7958x; 1.6127x over previous
import functools

import jax
import jax.numpy as jnp
from jax import lax
from jax.experimental import pallas as pl
from jax.experimental.pallas import tpu as pltpu
from jax.experimental.pallas import tpu_sc as plsc

B = 16384
H = 50
D = 32
NC = 2
NS = 16
NW = NC * NS
ROWS_W = B // NW
S = 8
N_ROUNDS = ROWS_W // S

_mesh = plsc.VectorSubcoreMesh(core_axis_name="c", subcore_axis_name="s")


@functools.partial(
    pl.kernel,
    out_type=jax.ShapeDtypeStruct((B, H, D), jnp.float32),
    mesh=_mesh,
    compiler_params=pltpu.CompilerParams(use_tc_tiling_on_sc=False),
    scratch_types=[
        pltpu.VMEM((ROWS_W, H), jnp.int32),
        pltpu.VMEM((S, 2, 1, H, D), jnp.float32),
        pltpu.SemaphoreType.DMA((S,)),
        pltpu.SemaphoreType.DMA((S,)),
    ],
)
def _gather(idx_hbm, table_hbm, out_hbm, idx_v, buf, gsem, ssem):
    wid = lax.axis_index("s") * NC + lax.axis_index("c")
    base = wid * ROWS_W

    pltpu.sync_copy(idx_hbm.at[pl.ds(base, ROWS_W)], idx_v)

    def fire(r, half, s):
        pltpu.async_copy(
            table_hbm.at[idx_v.at[r]], buf.at[s].at[half].at[0], gsem.at[s]
        )

    def wait_gather(r, half, s):
        pltpu.make_async_copy(
            table_hbm.at[idx_v.at[r]], buf.at[s].at[half].at[0], gsem.at[s]
        ).wait()

    def store(r, half, s):
        pltpu.async_copy(
            buf.at[s].at[half], out_hbm.at[pl.ds(base + r, 1)], ssem.at[s]
        )

    def drain_one_store(half, s):
        pltpu.make_async_copy(
            table_hbm.at[pl.ds(0, H)], buf.at[s].at[half].at[0], ssem.at[s]
        ).wait()

    for s in range(S):
        fire(s, 0, s)

    for s in range(S):
        wait_gather(s, 0, s)
        store(s, 0, s)
        fire(s + S, 1, s)

    def round_body(r, carry):
        h = r % 2
        hn = 1 - h
        for s in range(S):
            j = r * S + s
            wait_gather(j, h, s)
            store(j, h, s)
            drain_one_store(hn, s)
            fire(j + S, hn, s)
        return carry

    lax.fori_loop(1, N_ROUNDS - 1, round_body, 0)

    hl = (N_ROUNDS - 1) % 2
    for s in range(S):
        j = (N_ROUNDS - 1) * S + s
        wait_gather(j, hl, s)
        store(j, hl, s)

    for s in range(S):
        drain_one_store(0, s)
        drain_one_store(1, s)


def kernel(input, weight):
    return _gather(input.astype(jnp.int32), weight)
